# baseline (speedup 1.0000x reference)
import functools

import jax
import jax.numpy as jnp
from jax.experimental import pallas as pl
from jax.experimental.pallas import tpu as pltpu


def _rnn_block_kernel(x_ref, h0_ref, wih_t_ref, whh_t_ref, out_ref, hn_ref,
                      h_s, xw_s, *, tb_steps, bc):
    t_idx = pl.program_id(0)

    @pl.when(t_idx == 0)
    def _():
        h_s[...] = h0_ref[...]

    b = x_ref.shape[0]
    i_dim = x_ref.shape[2]
    h_dim = whh_t_ref.shape[1]
    half = h_dim // 2
    n_chunks = b // bc
    m_rows = bc * tb_steps

    for mc in range(n_chunks):
        xc = x_ref[mc * bc:(mc + 1) * bc].reshape(m_rows, i_dim)
        addr = (mc % 2) * (m_rows // 4)
        for mxu in range(2):
            pltpu.matmul_push_rhs(
                wih_t_ref[0:256, mxu * half:mxu * half + 256],
                staging_register=0, mxu_index=mxu)
            pltpu.matmul_push_rhs(
                wih_t_ref[256:512, mxu * half:mxu * half + 256],
                staging_register=1, mxu_index=mxu)
            pltpu.matmul_acc_lhs(addr, xc[:, :256], mxu_index=mxu,
                                 load_staged_rhs=0)
            pltpu.matmul_acc_lhs(addr, xc[:, 256:], mxu_index=mxu,
                                 load_staged_rhs=1)
        for mxu in range(2):
            y = pltpu.matmul_pop(addr, (m_rows, 256), jnp.float32,
                                 mxu_index=mxu)
            xw_s[mc * bc:(mc + 1) * bc, :, mxu * half:mxu * half + 256] = (
                y.reshape(bc, tb_steps, 256))

    def body(t, carry):
        ha, hb = carry
        for mxu in range(2):
            pltpu.matmul_push_rhs(
                whh_t_ref[0:256, mxu * half:mxu * half + 256],
                staging_register=0, mxu_index=mxu)
            pltpu.matmul_push_rhs(
                whh_t_ref[256:512, mxu * half:mxu * half + 256],
                staging_register=1, mxu_index=mxu)
            pltpu.matmul_acc_lhs(0, ha, mxu_index=mxu, load_staged_rhs=0)
            pltpu.matmul_acc_lhs(0, hb, mxu_index=mxu, load_staged_rhs=1)
        y0 = pltpu.matmul_pop(0, (b, 256), jnp.float32, mxu_index=0)
        y1 = pltpu.matmul_pop(0, (b, 256), jnp.float32, mxu_index=1)
        ha_n = jnp.maximum(xw_s[:, t, :256] + y0, 0.0)
        hb_n = jnp.maximum(xw_s[:, t, 256:] + y1, 0.0)
        out_ref[:, t, :256] = ha_n
        out_ref[:, t, 256:] = hb_n
        return (ha_n, hb_n)

    h0v = h_s[...]
    ha, hb = jax.lax.fori_loop(0, tb_steps, body,
                               (h0v[:, :256], h0v[:, 256:]), unroll=16)
    h_s[:, :256] = ha
    h_s[:, 256:] = hb
    hn_ref[:, :256] = ha
    hn_ref[:, 256:] = hb


def kernel(x, h0, W_ih, W_hh):
    B, T, I = x.shape
    H = W_hh.shape[0]
    tb = 128 if T % 128 == 0 else T
    nt = T // tb
    bc = 4 if B % 8 == 0 else B

    wih_t = W_ih.T
    whh_t = W_hh.T
    h0_2d = h0[0]

    out, h_n = pl.pallas_call(
        functools.partial(_rnn_block_kernel, tb_steps=tb, bc=bc),
        out_shape=(
            jax.ShapeDtypeStruct((B, T, H), x.dtype),
            jax.ShapeDtypeStruct((B, H), x.dtype),
        ),
        grid=(nt,),
        in_specs=[
            pl.BlockSpec((B, tb, I), lambda t: (0, t, 0)),
            pl.BlockSpec((B, H), lambda t: (0, 0)),
            pl.BlockSpec((I, H), lambda t: (0, 0)),
            pl.BlockSpec((H, H), lambda t: (0, 0)),
        ],
        out_specs=(
            pl.BlockSpec((B, tb, H), lambda t: (0, t, 0)),
            pl.BlockSpec((B, H), lambda t: (0, 0)),
        ),
        scratch_shapes=[
            pltpu.VMEM((B, H), jnp.float32),
            pltpu.VMEM((B, tb, H), jnp.float32),
        ],
        compiler_params=pltpu.CompilerParams(
            dimension_semantics=("arbitrary",),
            vmem_limit_bytes=56 * 1024 * 1024,
        ),
        name="rnn_relu_xmxu",
    )(x, h0_2d, wih_t, whh_t)
    return out, h_n[None]


# speedup vs baseline: 1.0160x; 1.0160x over previous
import functools

import jax
import jax.numpy as jnp
from jax.experimental import pallas as pl
from jax.experimental.pallas import tpu as pltpu


def _rnn_block_kernel(x_ref, h0_ref, wih_t_ref, whh_t_ref, out_ref, hn_ref,
                      h_s, xw_s, *, tb_steps, bc):
    t_idx = pl.program_id(0)

    @pl.when(t_idx == 0)
    def _():
        h_s[...] = h0_ref[...]

    b = x_ref.shape[0]
    i_dim = x_ref.shape[2]
    h_dim = whh_t_ref.shape[1]
    half = h_dim // 2
    n_chunks = b // bc
    m_rows = bc * tb_steps

    for mc in range(n_chunks):
        xc = x_ref[mc * bc:(mc + 1) * bc].reshape(m_rows, i_dim)
        addr = (mc % 2) * (m_rows // 4)
        for mxu in range(2):
            pltpu.matmul_push_rhs(
                wih_t_ref[0:256, mxu * half:mxu * half + 256],
                staging_register=0, mxu_index=mxu)
            pltpu.matmul_push_rhs(
                wih_t_ref[256:512, mxu * half:mxu * half + 256],
                staging_register=1, mxu_index=mxu)
            pltpu.matmul_acc_lhs(addr, xc[:, :256], mxu_index=mxu,
                                 load_staged_rhs=0)
            pltpu.matmul_acc_lhs(addr, xc[:, 256:], mxu_index=mxu,
                                 load_staged_rhs=1)
        for mxu in range(2):
            y = pltpu.matmul_pop(addr, (m_rows, 256), jnp.float32,
                                 mxu_index=mxu)
            xw_s[mc * bc:(mc + 1) * bc, :, mxu * half:mxu * half + 256] = (
                y.reshape(bc, tb_steps, 256))

    def _push_whh():
        for mxu in range(2):
            pltpu.matmul_push_rhs(
                whh_t_ref[0:256, mxu * half:mxu * half + 256],
                staging_register=0, mxu_index=mxu)
            pltpu.matmul_push_rhs(
                whh_t_ref[256:512, mxu * half:mxu * half + 256],
                staging_register=1, mxu_index=mxu)

    def _step(t, ha, hb, push_next):
        for mxu in range(2):
            pltpu.matmul_acc_lhs(0, ha, mxu_index=mxu, load_staged_rhs=0)
            pltpu.matmul_acc_lhs(0, hb, mxu_index=mxu, load_staged_rhs=1)
        if push_next:
            _push_whh()
        y0 = pltpu.matmul_pop(0, (b, 256), jnp.float32, mxu_index=0)
        y1 = pltpu.matmul_pop(0, (b, 256), jnp.float32, mxu_index=1)
        ha_n = jnp.maximum(xw_s[:, t, :256] + y0, 0.0)
        hb_n = jnp.maximum(xw_s[:, t, 256:] + y1, 0.0)
        out_ref[:, t, :256] = ha_n
        out_ref[:, t, 256:] = hb_n
        return ha_n, hb_n

    def body(t, carry):
        ha, hb = carry
        return _step(t, ha, hb, push_next=True)

    _push_whh()
    h0v = h_s[...]
    ha, hb = jax.lax.fori_loop(0, tb_steps - 1, body,
                               (h0v[:, :256], h0v[:, 256:]), unroll=16)
    ha, hb = _step(tb_steps - 1, ha, hb, push_next=False)
    h_s[:, :256] = ha
    h_s[:, 256:] = hb
    hn_ref[:, :256] = ha
    hn_ref[:, 256:] = hb


def kernel(x, h0, W_ih, W_hh):
    B, T, I = x.shape
    H = W_hh.shape[0]
    tb = 128 if T % 128 == 0 else T
    nt = T // tb
    bc = 4 if B % 8 == 0 else B

    wih_t = W_ih.T
    whh_t = W_hh.T
    h0_2d = h0[0]

    out, h_n = pl.pallas_call(
        functools.partial(_rnn_block_kernel, tb_steps=tb, bc=bc),
        out_shape=(
            jax.ShapeDtypeStruct((B, T, H), x.dtype),
            jax.ShapeDtypeStruct((B, H), x.dtype),
        ),
        grid=(nt,),
        in_specs=[
            pl.BlockSpec((B, tb, I), lambda t: (0, t, 0)),
            pl.BlockSpec((B, H), lambda t: (0, 0)),
            pl.BlockSpec((I, H), lambda t: (0, 0)),
            pl.BlockSpec((H, H), lambda t: (0, 0)),
        ],
        out_specs=(
            pl.BlockSpec((B, tb, H), lambda t: (0, t, 0)),
            pl.BlockSpec((B, H), lambda t: (0, 0)),
        ),
        scratch_shapes=[
            pltpu.VMEM((B, H), jnp.float32),
            pltpu.VMEM((B, tb, H), jnp.float32),
        ],
        compiler_params=pltpu.CompilerParams(
            dimension_semantics=("arbitrary",),
            vmem_limit_bytes=56 * 1024 * 1024,
        ),
        name="rnn_relu_xmxu",
    )(x, h0_2d, wih_t, whh_t)
    return out, h_n[None]


# speedup vs baseline: 1.0192x; 1.0031x over previous
import functools

import jax
import jax.numpy as jnp
from jax.experimental import pallas as pl
from jax.experimental.pallas import tpu as pltpu


def _rnn_block_kernel(x_ref, h0_ref, wih_t_ref, whh_t_ref, out_ref, hn_ref,
                      h_s, xw_s, *, tb_steps, bc):
    t_idx = pl.program_id(0)

    @pl.when(t_idx == 0)
    def _():
        h_s[...] = h0_ref[...]

    b = x_ref.shape[0]
    i_dim = x_ref.shape[2]
    h_dim = whh_t_ref.shape[1]
    half = h_dim // 2
    n_chunks = b // bc
    m_rows = bc * tb_steps

    for mc in range(n_chunks):
        xc = x_ref[mc * bc:(mc + 1) * bc].reshape(m_rows, i_dim)
        addr = (mc % 2) * (m_rows // 4)
        for mxu in range(2):
            pltpu.matmul_push_rhs(
                wih_t_ref[0:256, mxu * half:mxu * half + 256],
                staging_register=0, mxu_index=mxu)
            pltpu.matmul_push_rhs(
                wih_t_ref[256:512, mxu * half:mxu * half + 256],
                staging_register=1, mxu_index=mxu)
            pltpu.matmul_acc_lhs(addr, xc[:, :256], mxu_index=mxu,
                                 load_staged_rhs=0)
            pltpu.matmul_acc_lhs(addr, xc[:, 256:], mxu_index=mxu,
                                 load_staged_rhs=1)
        for mxu in range(2):
            y = pltpu.matmul_pop(addr, (m_rows, 256), jnp.float32,
                                 mxu_index=mxu)
            xw_s[mc * bc:(mc + 1) * bc, :, mxu * half:mxu * half + 256] = (
                y.reshape(bc, tb_steps, 256))

    def _push_whh():
        for mxu in range(2):
            pltpu.matmul_push_rhs(
                whh_t_ref[0:256, mxu * half:mxu * half + 256],
                staging_register=0, mxu_index=mxu)
            pltpu.matmul_push_rhs(
                whh_t_ref[256:512, mxu * half:mxu * half + 256],
                staging_register=1, mxu_index=mxu)

    bh = b // 2
    addr_b = 2 * ((bh // 4 + 1) // 2)

    def _step(t, h4, push_next):
        haa, hba, hab, hbb = h4
        for mxu in range(2):
            pltpu.matmul_acc_lhs(0, haa, mxu_index=mxu, load_staged_rhs=0)
            pltpu.matmul_acc_lhs(addr_b, hab, mxu_index=mxu,
                                 load_staged_rhs=None)
            pltpu.matmul_acc_lhs(0, hba, mxu_index=mxu, load_staged_rhs=1)
            pltpu.matmul_acc_lhs(addr_b, hbb, mxu_index=mxu,
                                 load_staged_rhs=None)
        if push_next:
            _push_whh()
        ya0 = pltpu.matmul_pop(0, (bh, 256), jnp.float32, mxu_index=0)
        ya1 = pltpu.matmul_pop(0, (bh, 256), jnp.float32, mxu_index=1)
        yb0 = pltpu.matmul_pop(addr_b, (bh, 256), jnp.float32, mxu_index=0)
        yb1 = pltpu.matmul_pop(addr_b, (bh, 256), jnp.float32, mxu_index=1)
        haa_n = jnp.maximum(xw_s[:bh, t, :256] + ya0, 0.0)
        hba_n = jnp.maximum(xw_s[:bh, t, 256:] + ya1, 0.0)
        hab_n = jnp.maximum(xw_s[bh:, t, :256] + yb0, 0.0)
        hbb_n = jnp.maximum(xw_s[bh:, t, 256:] + yb1, 0.0)
        out_ref[:bh, t, :256] = haa_n
        out_ref[:bh, t, 256:] = hba_n
        out_ref[bh:, t, :256] = hab_n
        out_ref[bh:, t, 256:] = hbb_n
        return (haa_n, hba_n, hab_n, hbb_n)

    def body(t, carry):
        return _step(t, carry, push_next=True)

    _push_whh()
    h0v = h_s[...]
    h4 = (h0v[:bh, :256], h0v[:bh, 256:], h0v[bh:, :256], h0v[bh:, 256:])
    h4 = jax.lax.fori_loop(0, tb_steps - 1, body, h4, unroll=16)
    haa, hba, hab, hbb = _step(tb_steps - 1, h4, push_next=False)
    h_s[:bh, :256] = haa
    h_s[:bh, 256:] = hba
    h_s[bh:, :256] = hab
    h_s[bh:, 256:] = hbb
    hn_ref[:bh, :256] = haa
    hn_ref[:bh, 256:] = hba
    hn_ref[bh:, :256] = hab
    hn_ref[bh:, 256:] = hbb


def kernel(x, h0, W_ih, W_hh):
    B, T, I = x.shape
    H = W_hh.shape[0]
    tb = 128 if T % 128 == 0 else T
    nt = T // tb
    bc = 4 if B % 8 == 0 else B

    wih_t = W_ih.T
    whh_t = W_hh.T
    h0_2d = h0[0]

    out, h_n = pl.pallas_call(
        functools.partial(_rnn_block_kernel, tb_steps=tb, bc=bc),
        out_shape=(
            jax.ShapeDtypeStruct((B, T, H), x.dtype),
            jax.ShapeDtypeStruct((B, H), x.dtype),
        ),
        grid=(nt,),
        in_specs=[
            pl.BlockSpec((B, tb, I), lambda t: (0, t, 0)),
            pl.BlockSpec((B, H), lambda t: (0, 0)),
            pl.BlockSpec((I, H), lambda t: (0, 0)),
            pl.BlockSpec((H, H), lambda t: (0, 0)),
        ],
        out_specs=(
            pl.BlockSpec((B, tb, H), lambda t: (0, t, 0)),
            pl.BlockSpec((B, H), lambda t: (0, 0)),
        ),
        scratch_shapes=[
            pltpu.VMEM((B, H), jnp.float32),
            pltpu.VMEM((B, tb, H), jnp.float32),
        ],
        compiler_params=pltpu.CompilerParams(
            dimension_semantics=("arbitrary",),
            vmem_limit_bytes=56 * 1024 * 1024,
        ),
        name="rnn_relu_xmxu",
    )(x, h0_2d, wih_t, whh_t)
    return out, h_n[None]
